# baseline (speedup 1.0000x reference)
import functools

import jax
import jax.numpy as jnp
from jax import lax
from jax.experimental import pallas as pl
from jax.experimental.pallas import tpu as pltpu
from jax.experimental.pallas import tpu_sc as plsc

B, L, D = 4096, 50, 16
NV, NU = 1000000, 1000000
NC, NS = 2, 16
NW = NC * NS
RPW = B * L // NW
CHUNK = 128
NCH = RPW // CHUNK
UPW = B // NW
BBLK = 256
LD = L * D


def _sc_gather(hist2d, nodes2d, v2e, u2e):
    mesh = plsc.VectorSubcoreMesh(core_axis_name="c", subcore_axis_name="s")

    @functools.partial(
        pl.kernel,
        mesh=mesh,
        out_type=(
            jax.ShapeDtypeStruct((NW * NCH, CHUNK, D), jnp.float32),
            jax.ShapeDtypeStruct((NW, CHUNK, D), jnp.float32),
        ),
        scratch_types=[
            pltpu.VMEM((NCH, CHUNK), jnp.int32),
            pltpu.VMEM((NCH, CHUNK, D), jnp.float32),
            pltpu.VMEM((1, CHUNK), jnp.int32),
            pltpu.VMEM((1, CHUNK, D), jnp.float32),
            pltpu.SemaphoreType.DMA,
            pltpu.SemaphoreType.DMA,
        ],
        compiler_params=pltpu.CompilerParams(use_tc_tiling_on_sc=False),
    )
    def k(hist_hbm, nodes_hbm, v2e_hbm, u2e_hbm, outv_hbm, outu_hbm,
          idx_v, rows_v, uidx_v, urows_v, sem, usem):
        wid = lax.axis_index("s") * NC + lax.axis_index("c")
        pltpu.sync_copy(hist_hbm.at[wid], idx_v)
        pltpu.sync_copy(nodes_hbm.at[wid], uidx_v)
        uh = pltpu.async_copy(u2e_hbm.at[uidx_v.at[0]], urows_v.at[0], usem)
        handles = [
            pltpu.async_copy(v2e_hbm.at[idx_v.at[j]], rows_v.at[j], sem)
            for j in range(NCH)
        ]
        uh.wait()
        pltpu.sync_copy(urows_v, outu_hbm.at[pl.ds(wid, 1)])
        for h in handles:
            h.wait()
        pltpu.sync_copy(rows_v, outv_hbm.at[pl.ds(wid * NCH, NCH)])

    return k(hist2d, nodes2d, v2e, u2e)


DTC = 8192
NP = 123 * DTC


def _tc_detile(vT, uT):
    grid = (NP // DTC,)
    f32 = jnp.float32

    def body(xv_ref, xu_ref, ov_ref, ou_ref):
        lane = lax.broadcasted_iota(jnp.int32, (D, 128), 1)
        sub = lax.broadcasted_iota(jnp.int32, (D, 128), 0)

        def detile(x, o_ref):
            acc = jnp.zeros((DTC // 8, 128), f32)
            for a in range(8):
                g = (lane == 16 * a + sub).astype(f32)
                acc = acc + lax.dot_general(
                    x[:, 1024 * a:1024 * (a + 1)], g,
                    (((0,), (0,)), ((), ())),
                    precision=lax.Precision.HIGHEST,
                    preferred_element_type=f32)
            o_ref[...] = acc

        detile(xv_ref[...], ov_ref)
        detile(xu_ref[...], ou_ref)

    return pl.pallas_call(
        body,
        grid=grid,
        in_specs=[
            pl.BlockSpec((D, DTC), lambda i: (0, i)),
            pl.BlockSpec((D, DTC), lambda i: (0, i)),
        ],
        out_specs=[
            pl.BlockSpec((DTC // 8, 128), lambda i: (i, 0)),
            pl.BlockSpec((DTC // 8, 128), lambda i: (i, 0)),
        ],
        out_shape=[
            jax.ShapeDtypeStruct((NP * D // 128, 128), jnp.float32),
            jax.ShapeDtypeStruct((NP * D // 128, 128), jnp.float32),
        ],
    )(vT, uT)


def _remap_idx(idx):
    return (idx & ~8191) + ((idx & 1023) << 3) + ((idx >> 10) & 7)


def _tc_body(x_ref, lab_ref, u_ref,
             w1a_ref, t5_ref, rep5_ref, w2_ref, b2_ref,
             a1o_ref, a1u_ref, ba1_ref, tile16_ref,
             a2_ref, ba2_ref, a3_ref, k50_ref, p16_ref,
             o_ref):
    f32 = jnp.float32
    x = x_ref[...]
    lab = lab_ref[...].astype(f32)
    lab_rep = jnp.dot(lab, rep5_ref[...], preferred_element_type=f32)
    r_iota = lax.broadcasted_iota(jnp.int32, (BBLK, 5 * L), 1) % 5
    oh = (jnp.abs(lab_rep - r_iota.astype(f32)) < 0.5).astype(f32)
    labterm = jnp.dot(oh, t5_ref[...], preferred_element_type=f32)
    h = jnp.maximum(jnp.dot(x, w1a_ref[...], preferred_element_type=f32) + labterm, 0.0)
    o = jnp.maximum(jnp.dot(h, w2_ref[...], preferred_element_type=f32) + b2_ref[...], 0.0)
    u = u_ref[...]
    uterm = jnp.dot(u, a1u_ref[...], preferred_element_type=f32) + ba1_ref[...]
    uterm = jnp.dot(uterm, tile16_ref[...], preferred_element_type=f32)
    a1 = jnp.maximum(jnp.dot(o, a1o_ref[...], preferred_element_type=f32) + uterm, 0.0)
    a2 = jnp.maximum(jnp.dot(a1, a2_ref[...], preferred_element_type=f32) + ba2_ref[...], 0.0)
    scores = jnp.dot(a2, a3_ref[...], preferred_element_type=f32)
    m = jnp.max(scores, axis=1, keepdims=True)
    e = jnp.exp(scores - m)
    att = e / jnp.sum(e, axis=1, keepdims=True)
    attrep = jnp.dot(att, k50_ref[...], preferred_element_type=f32)
    ah = jnp.dot(o * attrep, p16_ref[...], preferred_element_type=f32)
    o_ref[...] = (ah + u) * 0.5


def kernel(nodes_u, hist_ids, hist_labels, v2e, u2e, r2e,
           w_r1_w, w_r1_b, w_r2_w, w_r2_b,
           att1_w, att1_b, att2_w, att2_b, att3_w, att3_b):
    f32 = jnp.float32
    hist2d = _remap_idx(hist_ids.astype(jnp.int32)).reshape(NW, NCH, CHUNK)
    nodes2d = _remap_idx(nodes_u.astype(jnp.int32)).reshape(NW, 1, CHUNK)

    vlin, ulin = _tc_detile(v2e.T, u2e.T)
    outv, outu = _sc_gather(hist2d, nodes2d,
                            vlin.reshape(NP, D), ulin.reshape(NP, D))
    x800 = outv.reshape(B, LD)
    urep = outu.reshape(B, D)
    labels = hist_labels.astype(jnp.int32)

    eye_l = jnp.eye(L, dtype=f32)
    w1a = jnp.kron(eye_l, w_r1_w[:, :D].T)
    lab_tab = r2e @ w_r1_w[:, D:].T + w_r1_b[None, :]
    t5 = jnp.kron(eye_l, lab_tab)
    rep5 = jnp.kron(eye_l, jnp.ones((1, 5), dtype=f32))
    w2 = jnp.kron(eye_l, w_r2_w.T)
    b2 = jnp.tile(w_r2_b, L)[None, :]
    a1o = jnp.kron(eye_l, att1_w[:, :D].T)
    a1u = att1_w[:, D:].T
    ba1 = att1_b[None, :]
    tile16 = jnp.tile(jnp.eye(D, dtype=f32), (1, L))
    a2m = jnp.kron(eye_l, att2_w.T)
    ba2 = jnp.tile(att2_b, L)[None, :]
    a3 = jnp.kron(eye_l, att3_w.T)
    k50 = jnp.kron(eye_l, jnp.ones((1, D), dtype=f32))
    p16 = jnp.tile(jnp.eye(D, dtype=f32), (L, 1))

    grid = (B // BBLK,)

    def blk(shape):
        return pl.BlockSpec(shape, lambda i: (0, 0))

    out = pl.pallas_call(
        _tc_body,
        grid=grid,
        in_specs=[
            pl.BlockSpec((BBLK, LD), lambda i: (i, 0)),
            pl.BlockSpec((BBLK, L), lambda i: (i, 0)),
            pl.BlockSpec((BBLK, D), lambda i: (i, 0)),
            blk((LD, LD)), blk((5 * L, LD)), blk((L, 5 * L)),
            blk((LD, LD)), blk((1, LD)),
            blk((LD, LD)), blk((D, D)), blk((1, D)), blk((D, LD)),
            blk((LD, LD)), blk((1, LD)), blk((LD, L)),
            blk((L, LD)), blk((LD, D)),
        ],
        out_specs=pl.BlockSpec((BBLK, D), lambda i: (i, 0)),
        out_shape=jax.ShapeDtypeStruct((B, D), f32),
    )(x800, labels, urep,
      w1a, t5, rep5, w2, b2,
      a1o, a1u, ba1, tile16,
      a2m, ba2, a3, k50, p16)
    return out


# speedup vs baseline: 3.5887x; 3.5887x over previous
import functools

import jax
import jax.numpy as jnp
from jax import lax
from jax.experimental import pallas as pl
from jax.experimental.pallas import tpu as pltpu
from jax.experimental.pallas import tpu_sc as plsc

B, L, D = 4096, 50, 16
NV, NU = 1000000, 1000000
NC, NS = 2, 16
NW = NC * NS
RPW = B * L // NW
CHUNK = 128
NCH = RPW // CHUNK
UPW = B // NW
BBLK = 256
LD = L * D


def _sc_gather(hist2d, nodes2d, v2e, u2e):
    mesh = plsc.VectorSubcoreMesh(core_axis_name="c", subcore_axis_name="s")

    @functools.partial(
        pl.kernel,
        mesh=mesh,
        out_type=(
            jax.ShapeDtypeStruct((NW * NCH, CHUNK, D), jnp.float32),
            jax.ShapeDtypeStruct((NW, CHUNK, D), jnp.float32),
        ),
        scratch_types=[
            pltpu.VMEM((NCH, CHUNK), jnp.int32),
            pltpu.VMEM((NCH, CHUNK, D), jnp.float32),
            pltpu.VMEM((1, CHUNK), jnp.int32),
            pltpu.VMEM((1, CHUNK, D), jnp.float32),
            pltpu.SemaphoreType.DMA,
            pltpu.SemaphoreType.DMA,
        ],
        compiler_params=pltpu.CompilerParams(use_tc_tiling_on_sc=False),
    )
    def k(hist_hbm, nodes_hbm, v2e_hbm, u2e_hbm, outv_hbm, outu_hbm,
          idx_v, rows_v, uidx_v, urows_v, sem, usem):
        wid = lax.axis_index("s") * NC + lax.axis_index("c")
        pltpu.sync_copy(hist_hbm.at[wid], idx_v)
        pltpu.sync_copy(nodes_hbm.at[wid], uidx_v)
        uh = pltpu.async_copy(u2e_hbm.at[uidx_v.at[0]], urows_v.at[0], usem)
        handles = [
            pltpu.async_copy(v2e_hbm.at[idx_v.at[j]], rows_v.at[j], sem)
            for j in range(NCH)
        ]
        uh.wait()
        pltpu.sync_copy(urows_v, outu_hbm.at[pl.ds(wid, 1)])
        for h in handles:
            h.wait()
        pltpu.sync_copy(rows_v, outv_hbm.at[pl.ds(wid * NCH, NCH)])

    return k(hist2d, nodes2d, v2e, u2e)


DTC = 1 << 16
NP = 16 * DTC
DCH = DTC // 8


def _tc_detile(vT, uT):
    grid = (NP // DTC,)

    def body(xv_ref, xu_ref, ov_ref, ou_ref):
        def detile(x, o_ref):
            xb = jnp.concatenate(
                [x[:, a * DCH:(a + 1) * DCH] for a in range(8)], axis=0)
            o_ref[...] = xb.T

        detile(xv_ref[...], ov_ref)
        detile(xu_ref[...], ou_ref)

    return pl.pallas_call(
        body,
        grid=grid,
        in_specs=[
            pl.BlockSpec((D, DTC), lambda i: (0, i)),
            pl.BlockSpec((D, DTC), lambda i: (0, i)),
        ],
        out_specs=[
            pl.BlockSpec((DTC // 8, 128), lambda i: (i, 0)),
            pl.BlockSpec((DTC // 8, 128), lambda i: (i, 0)),
        ],
        out_shape=[
            jax.ShapeDtypeStruct((NP * D // 128, 128), jnp.float32),
            jax.ShapeDtypeStruct((NP * D // 128, 128), jnp.float32),
        ],
    )(vT, uT)


def _remap_idx(idx):
    return (idx & ~(DTC - 1)) + ((idx & (DCH - 1)) << 3) + ((idx // DCH) & 7)


def _tc_body(x_ref, lab_ref, u_ref,
             w1a_ref, t5_ref, rep5_ref, w2_ref, b2_ref,
             a1o_ref, a1u_ref, ba1_ref, tile16_ref,
             a2_ref, ba2_ref, a3_ref, k50_ref, p16_ref,
             o_ref):
    f32 = jnp.float32
    x = x_ref[...]
    lab = lab_ref[...].astype(f32)
    lab_rep = jnp.dot(lab, rep5_ref[...], preferred_element_type=f32)
    r_iota = lax.broadcasted_iota(jnp.int32, (BBLK, 5 * L), 1) % 5
    oh = (jnp.abs(lab_rep - r_iota.astype(f32)) < 0.5).astype(f32)
    labterm = jnp.dot(oh, t5_ref[...], preferred_element_type=f32)
    h = jnp.maximum(jnp.dot(x, w1a_ref[...], preferred_element_type=f32) + labterm, 0.0)
    o = jnp.maximum(jnp.dot(h, w2_ref[...], preferred_element_type=f32) + b2_ref[...], 0.0)
    u = u_ref[...]
    uterm = jnp.dot(u, a1u_ref[...], preferred_element_type=f32) + ba1_ref[...]
    uterm = jnp.dot(uterm, tile16_ref[...], preferred_element_type=f32)
    a1 = jnp.maximum(jnp.dot(o, a1o_ref[...], preferred_element_type=f32) + uterm, 0.0)
    a2 = jnp.maximum(jnp.dot(a1, a2_ref[...], preferred_element_type=f32) + ba2_ref[...], 0.0)
    scores = jnp.dot(a2, a3_ref[...], preferred_element_type=f32)
    m = jnp.max(scores, axis=1, keepdims=True)
    e = jnp.exp(scores - m)
    att = e / jnp.sum(e, axis=1, keepdims=True)
    attrep = jnp.dot(att, k50_ref[...], preferred_element_type=f32)
    ah = jnp.dot(o * attrep, p16_ref[...], preferred_element_type=f32)
    o_ref[...] = (ah + u) * 0.5


def kernel(nodes_u, hist_ids, hist_labels, v2e, u2e, r2e,
           w_r1_w, w_r1_b, w_r2_w, w_r2_b,
           att1_w, att1_b, att2_w, att2_b, att3_w, att3_b):
    f32 = jnp.float32
    hist2d = _remap_idx(hist_ids.astype(jnp.int32)).reshape(NW, NCH, CHUNK)
    nodes2d = _remap_idx(nodes_u.astype(jnp.int32)).reshape(NW, 1, CHUNK)

    vlin, ulin = _tc_detile(v2e.T, u2e.T)
    outv, outu = _sc_gather(hist2d, nodes2d,
                            vlin.reshape(NP, D), ulin.reshape(NP, D))
    x800 = outv.reshape(B, LD)
    urep = outu.reshape(B, D)
    labels = hist_labels.astype(jnp.int32)

    eye_l = jnp.eye(L, dtype=f32)
    w1a = jnp.kron(eye_l, w_r1_w[:, :D].T)
    lab_tab = r2e @ w_r1_w[:, D:].T + w_r1_b[None, :]
    t5 = jnp.kron(eye_l, lab_tab)
    rep5 = jnp.kron(eye_l, jnp.ones((1, 5), dtype=f32))
    w2 = jnp.kron(eye_l, w_r2_w.T)
    b2 = jnp.tile(w_r2_b, L)[None, :]
    a1o = jnp.kron(eye_l, att1_w[:, :D].T)
    a1u = att1_w[:, D:].T
    ba1 = att1_b[None, :]
    tile16 = jnp.tile(jnp.eye(D, dtype=f32), (1, L))
    a2m = jnp.kron(eye_l, att2_w.T)
    ba2 = jnp.tile(att2_b, L)[None, :]
    a3 = jnp.kron(eye_l, att3_w.T)
    k50 = jnp.kron(eye_l, jnp.ones((1, D), dtype=f32))
    p16 = jnp.tile(jnp.eye(D, dtype=f32), (L, 1))

    grid = (B // BBLK,)

    def blk(shape):
        return pl.BlockSpec(shape, lambda i: (0, 0))

    out = pl.pallas_call(
        _tc_body,
        grid=grid,
        in_specs=[
            pl.BlockSpec((BBLK, LD), lambda i: (i, 0)),
            pl.BlockSpec((BBLK, L), lambda i: (i, 0)),
            pl.BlockSpec((BBLK, D), lambda i: (i, 0)),
            blk((LD, LD)), blk((5 * L, LD)), blk((L, 5 * L)),
            blk((LD, LD)), blk((1, LD)),
            blk((LD, LD)), blk((D, D)), blk((1, D)), blk((D, LD)),
            blk((LD, LD)), blk((1, LD)), blk((LD, L)),
            blk((L, LD)), blk((LD, D)),
        ],
        out_specs=pl.BlockSpec((BBLK, D), lambda i: (i, 0)),
        out_shape=jax.ShapeDtypeStruct((B, D), f32),
    )(x800, labels, urep,
      w1a, t5, rep5, w2, b2,
      a1o, a1u, ba1, tile16,
      a2m, ba2, a3, k50, p16)
    return out
